# Initial kernel scaffold; baseline (speedup 1.0000x reference)
import jax
import jax.numpy as jnp
from jax.experimental import pallas as pl


def kernel(input, table):
    raise NotImplementedError("write your pallas kernel here")


# speedup vs baseline: 1.9115x; 1.9115x over previous
import functools

import jax
import jax.numpy as jnp
from jax import lax
from jax.experimental import pallas as pl
from jax.experimental.pallas import tpu as pltpu
from jax.experimental.pallas import tpu_sc as plsc

PAD = 1
SEQ = 4096
BATCH = 4
DIM = 1024
TOTAL = BATCH * SEQ
NUM_TILES = 32
TOK_PER_TILE = TOTAL // NUM_TILES
CHUNKS_PER_ROW = SEQ // TOK_PER_TILE
CH = 32
NCH = TOK_PER_TILE // CH
L = 16


def _body(inp_hbm, table_hbm, out_hbm, tokens_v, idx_v, buf0, buf1, sem0, sem1):
  nc = 2
  wid = lax.axis_index("s") * nc + lax.axis_index("c")
  row = wid // CHUNKS_PER_ROW
  chunk = wid % CHUNKS_PER_ROW
  rbase = row * SEQ

  pltpu.sync_copy(inp_hbm.at[pl.ds(rbase, SEQ)], tokens_v)

  nvecs = chunk * (TOK_PER_TILE // L)

  def obody(i, acc):
    v = tokens_v[pl.ds(i * L, L)]
    m = jnp.where(v != PAD, jnp.int32(1), jnp.int32(0))
    return acc + jnp.sum(m)

  offset = lax.fori_loop(0, nvecs, obody, jnp.int32(0))

  base = chunk * TOK_PER_TILE

  def cbody(i, carry):
    v = tokens_v[pl.ds(base + i * L, L)]
    m = jnp.where(v != PAD, jnp.int32(1), jnp.int32(0))
    cs = jnp.cumsum(m) + carry
    pos = jnp.where(v != PAD, cs, jnp.int32(0)) + PAD
    idx_v[pl.ds(i * L, L)] = pos
    return carry + jnp.sum(m)

  lax.fori_loop(0, TOK_PER_TILE // L, cbody, offset, unroll=2)

  bufs = (buf0, buf1)
  sems = (sem0, sem1)
  out_base = rbase + base
  handles = [None, None]
  handles[0] = pltpu.async_copy(
      table_hbm.at[idx_v.at[pl.ds(0, CH)]], bufs[0], sems[0])
  for g in range(NCH):
    if g + 1 < NCH:
      nb = (g + 1) % 2
      handles[nb] = pltpu.async_copy(
          table_hbm.at[idx_v.at[pl.ds((g + 1) * CH, CH)]], bufs[nb], sems[nb])
    handles[g % 2].wait()
    pltpu.sync_copy(bufs[g % 2], out_hbm.at[pl.ds(out_base + g * CH, CH)])


@jax.jit
def _lookup(inp_flat, table):
  mesh = plsc.VectorSubcoreMesh(core_axis_name="c", subcore_axis_name="s")
  k = functools.partial(
      pl.kernel,
      mesh=mesh,
      compiler_params=pltpu.CompilerParams(needs_layout_passes=False),
      out_type=jax.ShapeDtypeStruct((TOTAL, DIM), jnp.float32),
      scratch_types=[
          pltpu.VMEM((SEQ,), jnp.int32),
          pltpu.VMEM((TOK_PER_TILE,), jnp.int32),
          pltpu.VMEM((CH, DIM), jnp.float32),
          pltpu.VMEM((CH, DIM), jnp.float32),
          pltpu.SemaphoreType.DMA,
          pltpu.SemaphoreType.DMA,
      ],
  )(_body)
  return k(inp_flat, table)


def kernel(input, table):
  out = _lookup(input.reshape(-1), table)
  return out.reshape(BATCH, SEQ, DIM)


# speedup vs baseline: 1.9386x; 1.0142x over previous
import functools

import jax
import jax.numpy as jnp
from jax import lax
from jax.experimental import pallas as pl
from jax.experimental.pallas import tpu as pltpu
from jax.experimental.pallas import tpu_sc as plsc

PAD = 1
SEQ = 4096
BATCH = 4
DIM = 1024
TOTAL = BATCH * SEQ
NUM_TILES = 32
TOK_PER_TILE = TOTAL // NUM_TILES
CHUNKS_PER_ROW = SEQ // TOK_PER_TILE
CH = 32
NCH = TOK_PER_TILE // CH
L = 16


NBUF = 3


def _body(inp_hbm, table_hbm, out_hbm, tokens_v, idx_v,
          buf0, buf1, buf2, gs0, gs1, gs2, os0, os1, os2):
  nc = 2
  wid = lax.axis_index("s") * nc + lax.axis_index("c")
  row = wid // CHUNKS_PER_ROW
  chunk = wid % CHUNKS_PER_ROW
  rbase = row * SEQ

  pltpu.sync_copy(inp_hbm.at[pl.ds(rbase, SEQ)], tokens_v)

  nvecs = chunk * (TOK_PER_TILE // L)

  def obody(i, acc):
    v = tokens_v[pl.ds(i * L, L)]
    return acc + jnp.where(v != PAD, jnp.int32(1), jnp.int32(0))

  accv = lax.fori_loop(0, nvecs, obody, jnp.zeros((L,), jnp.int32))
  offset = jnp.sum(accv)

  base = chunk * TOK_PER_TILE

  def cbody(i, carry):
    v = tokens_v[pl.ds(base + i * L, L)]
    m = jnp.where(v != PAD, jnp.int32(1), jnp.int32(0))
    cs = jnp.cumsum(m) + carry
    pos = jnp.where(v != PAD, cs, jnp.int32(0)) + PAD
    idx_v[pl.ds(i * L, L)] = pos
    return cs[L - 1]

  lax.fori_loop(0, TOK_PER_TILE // L, cbody, offset, unroll=2)

  bufs = (buf0, buf1, buf2)
  gsems = (gs0, gs1, gs2)
  osems = (os0, os1, os2)
  out_base = rbase + base
  gh = [None] * NBUF
  oh = [None] * NBUF

  def fire_gather(g):
    b = g % NBUF
    gh[b] = pltpu.async_copy(
        table_hbm.at[idx_v.at[pl.ds(g * CH, CH)]], bufs[b], gsems[b])

  for g in range(min(NBUF - 1, NCH)):
    fire_gather(g)
  for g in range(NCH):
    b = g % NBUF
    gh[b].wait()
    oh[b] = pltpu.async_copy(
        bufs[b], out_hbm.at[pl.ds(out_base + g * CH, CH)], osems[b])
    nxt = g + NBUF - 1
    if nxt < NCH:
      nb = nxt % NBUF
      if oh[nb] is not None:
        oh[nb].wait()
      fire_gather(nxt)
  for g in range(max(0, NCH - (NBUF - 1)), NCH):
    oh[g % NBUF].wait()


@jax.jit
def _lookup(inp_flat, table):
  mesh = plsc.VectorSubcoreMesh(core_axis_name="c", subcore_axis_name="s")
  k = functools.partial(
      pl.kernel,
      mesh=mesh,
      compiler_params=pltpu.CompilerParams(needs_layout_passes=False),
      out_type=jax.ShapeDtypeStruct((TOTAL, DIM), jnp.float32),
      scratch_types=[
          pltpu.VMEM((SEQ,), jnp.int32),
          pltpu.VMEM((TOK_PER_TILE,), jnp.int32),
          pltpu.VMEM((CH, DIM), jnp.float32),
          pltpu.VMEM((CH, DIM), jnp.float32),
          pltpu.VMEM((CH, DIM), jnp.float32),
          pltpu.SemaphoreType.DMA,
          pltpu.SemaphoreType.DMA,
          pltpu.SemaphoreType.DMA,
          pltpu.SemaphoreType.DMA,
          pltpu.SemaphoreType.DMA,
          pltpu.SemaphoreType.DMA,
      ],
  )(_body)
  return k(inp_flat, table)


def kernel(input, table):
  out = _lookup(input.reshape(-1), table)
  return out.reshape(BATCH, SEQ, DIM)
